# baseline (speedup 1.0000x reference)
import jax
import jax.numpy as jnp
from jax import lax
from jax.experimental import pallas as pl
from jax.experimental.pallas import tpu as pltpu
from jax.experimental.pallas import tpu_sc as plsc

H_DIM = 128
N_HN = 100000
N_HE = 320000

NC = 2
NS = 16
NW = NC * NS

CHUNK = 128
NBUF = 5

N_CHUNKS_N = 25
N_CHUNKS_E = 80
N_PAD = NW * N_CHUNKS_N * CHUNK
E_PAD = NW * N_CHUNKS_E * CHUNK


def _emb_kernel(hn_hbm, he_hbm, n_table_hbm, e_table_hbm,
                n_out_hbm, e_out_hbm,
                idx_n, idx_e, rows, gsems, osems, isem):
    wid = lax.axis_index("s") * NC + lax.axis_index("c")

    nn = N_CHUNKS_N * CHUNK
    ne = N_CHUNKS_E * CHUNK
    pltpu.async_copy(hn_hbm.at[pl.ds(wid * nn, nn)], idx_n, isem)
    pltpu.make_async_copy(hn_hbm.at[pl.ds(0, nn)], idx_n, isem).wait()
    pltpu.async_copy(he_hbm.at[pl.ds(wid * ne, ne)], idx_e, isem)
    pltpu.make_async_copy(he_hbm.at[pl.ds(0, ne)], idx_e, isem).wait()

    def run_table(idx_v, table_hbm, out_hbm, n_chunks):
        base = wid * (n_chunks * CHUNK)
        n_outer = n_chunks // NBUF

        def start_gather(j, b):
            pltpu.async_copy(table_hbm.at[idx_v.at[pl.ds(j * CHUNK, CHUNK)]],
                             rows.at[b], gsems[b])

        def wait_gather(b):
            pltpu.make_async_copy(table_hbm.at[idx_v.at[pl.ds(0, CHUNK)]],
                                  rows.at[b], gsems[b]).wait()

        def start_store(j, b):
            pltpu.async_copy(rows.at[b], out_hbm.at[pl.ds(base + j * CHUNK,
                                                          CHUNK)], osems[b])

        def wait_store(b):
            pltpu.make_async_copy(rows.at[b],
                                  out_hbm.at[pl.ds(base, CHUNK)],
                                  osems[b]).wait()

        for b in range(NBUF):
            start_gather(b, b)

        def outer(g, _):
            j0 = g * NBUF
            for b in range(NBUF):
                j = j0 + b
                wait_gather(b)
                start_store(j, b)
                wait_store(b)

                @pl.when(g < n_outer - 1)
                def _():
                    start_gather(j + NBUF, b)
            return 0

        lax.fori_loop(0, n_outer, outer, 0)

    run_table(idx_n, n_table_hbm, n_out_hbm, N_CHUNKS_N)
    run_table(idx_e, e_table_hbm, e_out_hbm, N_CHUNKS_E)


@jax.jit
def _run(hn_pad, he_pad, n_table, e_table):
    mesh = plsc.VectorSubcoreMesh(core_axis_name="c", subcore_axis_name="s")
    f = pl.kernel(
        _emb_kernel,
        out_type=(
            jax.ShapeDtypeStruct((N_PAD, H_DIM), jnp.float32),
            jax.ShapeDtypeStruct((E_PAD, H_DIM), jnp.float32),
        ),
        mesh=mesh,
        scratch_types=[
            pltpu.VMEM((N_CHUNKS_N * CHUNK,), jnp.int32),
            pltpu.VMEM((N_CHUNKS_E * CHUNK,), jnp.int32),
            pltpu.VMEM((NBUF, CHUNK, H_DIM), jnp.float32),
            [pltpu.SemaphoreType.DMA] * NBUF,
            [pltpu.SemaphoreType.DMA] * NBUF,
            pltpu.SemaphoreType.DMA,
        ],
    )
    return f(hn_pad, he_pad, n_table, e_table)


def kernel(g, hn, r, he, norm, n_table, e_table):
    hn_flat = hn.reshape(-1).astype(jnp.int32)
    he_flat = he.reshape(-1).astype(jnp.int32)
    hn_pad = jnp.pad(hn_flat, (0, N_PAD - N_HN))
    he_pad = jnp.pad(he_flat, (0, E_PAD - N_HE))
    n_full, e_full = _run(hn_pad, he_pad, n_table, e_table)
    return (n_full[:N_HN], e_full[:N_HE])
